# baseline (speedup 1.0000x reference)
import math

import jax
import jax.numpy as jnp
from jax import lax
from jax.experimental import pallas as pl
from jax.experimental.pallas import tpu as pltpu
from jax.experimental.pallas import tpu_sc as plsc

B, N, C = 4, 2048, 768
BN = B * N
NW = 32
RPW = BN // NW
GCH = 64
NCH = RPW // GCH
TM = 512
NPB = N // TM
F32 = jnp.float32
BF16 = jnp.bfloat16


def _prep_body(x_ref, wq_ref, wk_ref, wv_ref, wo_ref, bq_ref, bk_ref,
               bv_ref, rb_ref, m_ref, wvo_ref, w_ref, bvo_ref, c1_ref,
               pm_ref, xu_ref, u_s):
    @pl.when(pl.program_id(0) == 0)
    def _weights():
        wq = wq_ref[...]
        wk = wk_ref[...]
        wo = wo_ref[...]
        m_ref[...] = lax.dot_general(
            wq, wk, (((0,), (0,)), ((), ())),
            preferred_element_type=F32).astype(BF16)
        wvo_ref[...] = lax.dot_general(
            wo, wv_ref[...], (((1,), (0,)), ((), ())),
            preferred_element_type=F32).astype(BF16)
        u_s[...] = lax.dot_general(
            bk_ref[...], wq, (((1,), (0,)), ((), ())),
            preferred_element_type=F32)
        w_ref[...] = lax.dot_general(
            bq_ref[...], wk, (((1,), (0,)), ((), ())),
            preferred_element_type=F32)
        bvo_ref[...] = lax.dot_general(
            bv_ref[...], wo, (((1,), (1,)), ((), ())),
            preferred_element_type=F32)
        c1_ref[...] = (jnp.sum(bq_ref[...] * bk_ref[...], axis=1,
                               keepdims=True) + rb_ref[...])

    xb = x_ref[...]
    pm_ref[...] = lax.dot_general(
        xb.astype(BF16), m_ref[...], (((1,), (0,)), ((), ())),
        preferred_element_type=F32).astype(BF16)
    xu_ref[...] = jnp.sum(xb * u_s[...], axis=1, keepdims=True)


def _sc_gather_body(x2d_hbm, nb_hbm, xg_hbm, xg2_hbm,
                    nb_v, idxf_v, idx2f_v, buf0, buf1,
                    semg0, semg1, sems0, sems1):
    nc = plsc.get_sparse_core_info().num_cores
    wid = lax.axis_index("s") * nc + lax.axis_index("c")
    base = wid * RPW
    b = base // N
    i0 = base - b * N
    bN = b * N

    pltpu.sync_copy(nb_hbm, nb_v)

    for k in range(RPW // 16):
        c, o = k // (GCH // 16), (k % (GCH // 16)) * 16
        iv = nb_v[pl.ds(i0 + k * 16, 16)]
        i2v = plsc.load_gather(nb_v, [iv])
        idxf_v[c, pl.ds(o, 16)] = iv + bN
        idx2f_v[c, pl.ds(o, 16)] = i2v + bN

    chunks = ([(idxf_v.at[c], xg_hbm, c) for c in range(NCH)]
              + [(idx2f_v.at[c], xg2_hbm, c) for c in range(NCH)])
    bufs = (buf0, buf1)
    sems = (sems0, sems1)
    stores = [None] * len(chunks)
    for j, (idx_ref, out_hbm, c) in enumerate(chunks):
        bi = j & 1
        if j >= 2:
            stores[j - 2].wait()
        pltpu.async_copy(x2d_hbm.at[idx_ref], bufs[bi], semg0).wait()
        stores[j] = pltpu.async_copy(
            bufs[bi], out_hbm.at[pl.ds(base + c * GCH, GCH)], sems[bi])
    stores[-2].wait()
    stores[-1].wait()


def _sc_gather(x2d, nb1d):
    mesh = plsc.VectorSubcoreMesh(core_axis_name="c", subcore_axis_name="s")
    f = pl.kernel(
        _sc_gather_body,
        out_type=[jax.ShapeDtypeStruct((BN, C), F32),
                  jax.ShapeDtypeStruct((BN, C), F32)],
        mesh=mesh,
        scratch_types=[
            pltpu.VMEM((N,), jnp.int32),
            pltpu.VMEM((NCH, GCH), jnp.int32),
            pltpu.VMEM((NCH, GCH), jnp.int32),
            pltpu.VMEM((GCH, C), F32),
            pltpu.VMEM((GCH, C), F32),
            pltpu.SemaphoreType.DMA,
            pltpu.SemaphoreType.DMA,
            pltpu.SemaphoreType.DMA,
            pltpu.SemaphoreType.DMA,
        ],
        compiler_params=pltpu.CompilerParams(needs_layout_passes=False),
    )
    return f(x2d, nb1d)


def _main_body(pm_ref, xu_ref, xg_ref, xg2_ref, wvo_ref, w_ref,
               bvo_ref, c1_ref, bo_ref, out_ref, cs_s, so_s):
    p = pl.program_id(0)
    i = pl.program_id(1)
    b = i // NPB

    @pl.when((p == 0) & (i == 0))
    def _zero():
        cs_s[...] = jnp.zeros_like(cs_s)

    @pl.when(p == 0)
    def _colsum():
        cs_s[pl.ds(b, 1), :] += jnp.sum(xg_ref[...], axis=0, keepdims=True)

    @pl.when((p == 1) & (i == 0))
    def _so():
        so_s[...] = lax.dot_general(
            cs_s[...].astype(BF16), wvo_ref[...], (((1,), (1,)), ((), ())),
            preferred_element_type=F32) + float(N) * bvo_ref[...]

    @pl.when(p == 1)
    def _compute():
        xgb = xg_ref[...]
        s = (jnp.sum(pm_ref[...].astype(F32) * xgb, axis=1, keepdims=True)
             + xu_ref[...]
             + jnp.sum(xgb * w_ref[...], axis=1, keepdims=True)
             + c1_ref[0, 0])
        e = jnp.exp(jnp.minimum(s * (1.0 / math.sqrt(C)), 80.0))
        z = e + (N - 1.0)
        g2o = lax.dot_general(
            xg2_ref[...].astype(BF16), wvo_ref[...], (((1,), (1,)), ((), ())),
            preferred_element_type=F32) + bvo_ref[...]
        sob = so_s[pl.ds(b, 1), :]
        out_ref[...] = (sob + (e - 1.0) * g2o) / z + bo_ref[...]


def _pin(shape):
    return pl.BlockSpec(shape, lambda i: (0, 0))


def _pin2(shape):
    return pl.BlockSpec(shape, lambda p, i: (0, 0))


def kernel(x, neighbors, Wq, bq, Wk, bk, Wv, bv, relative_bias, Wo, bo):
    x2d = x.reshape(BN, C)
    nb1d = neighbors[:, 0]

    xg, xg2 = _sc_gather(x2d, nb1d)

    m, wvo, w, bvo, c1, pm, xu = pl.pallas_call(
        _prep_body,
        grid=(BN // TM,),
        in_specs=[pl.BlockSpec((TM, C), lambda i: (i, 0)),
                  _pin((C, C)), _pin((C, C)), _pin((C, C)), _pin((C, C)),
                  _pin((1, C)), _pin((1, C)), _pin((1, C)), _pin((1, 1))],
        out_specs=[_pin((C, C)), _pin((C, C)), _pin((1, C)), _pin((1, C)),
                   _pin((1, 1)),
                   pl.BlockSpec((TM, C), lambda i: (i, 0)),
                   pl.BlockSpec((TM, 1), lambda i: (i, 0))],
        out_shape=[jax.ShapeDtypeStruct((C, C), BF16),
                   jax.ShapeDtypeStruct((C, C), BF16),
                   jax.ShapeDtypeStruct((1, C), F32),
                   jax.ShapeDtypeStruct((1, C), F32),
                   jax.ShapeDtypeStruct((1, 1), F32),
                   jax.ShapeDtypeStruct((BN, C), BF16),
                   jax.ShapeDtypeStruct((BN, 1), F32)],
        scratch_shapes=[pltpu.VMEM((1, C), F32)],
        compiler_params=pltpu.CompilerParams(
            dimension_semantics=("arbitrary",)),
    )(x2d, Wq, Wk, Wv, Wo, bq.reshape(1, C), bk.reshape(1, C),
      bv.reshape(1, C), relative_bias)

    row_p1 = pl.BlockSpec(
        (TM, C), lambda p, i: (jnp.where(p == 0, 0, i), 0))
    row_both = pl.BlockSpec((TM, C), lambda p, i: (i, 0))
    col_p1 = pl.BlockSpec(
        (TM, 1), lambda p, i: (jnp.where(p == 0, 0, i), 0))
    out2d = pl.pallas_call(
        _main_body,
        grid=(2, BN // TM),
        in_specs=[row_p1, col_p1, row_both, row_p1,
                  _pin2((C, C)), _pin2((1, C)), _pin2((1, C)),
                  _pin2((1, 1)), _pin2((1, C))],
        out_specs=row_p1,
        out_shape=jax.ShapeDtypeStruct((BN, C), F32),
        scratch_shapes=[pltpu.VMEM((B, C), F32), pltpu.VMEM((B, C), F32)],
        compiler_params=pltpu.CompilerParams(
            dimension_semantics=("arbitrary", "arbitrary")),
    )(pm, xu, xg, xg2, wvo, w, bvo, c1, bo.reshape(1, C))

    return out2d.reshape(B, N, C)


# speedup vs baseline: 1.0600x; 1.0600x over previous
import math

import jax
import jax.numpy as jnp
from jax import lax
from jax.experimental import pallas as pl
from jax.experimental.pallas import tpu as pltpu
from jax.experimental.pallas import tpu_sc as plsc

B, N, C = 4, 2048, 768
BN = B * N
NW = 32
RPW = BN // NW
GCH = 64
NCH = RPW // GCH
TM = 512
NPB = N // TM
NBLK = BN // TM
F32 = jnp.float32
BF16 = jnp.bfloat16


def _sc_gather_body(x2d_hbm, nb_hbm, xg_hbm, xg2_hbm,
                    nb_v, idxf_v, idx2f_v, buf0, buf1,
                    semg, sems0, sems1):
    nc = plsc.get_sparse_core_info().num_cores
    wid = lax.axis_index("s") * nc + lax.axis_index("c")
    base = wid * RPW
    b = base // N
    i0 = base - b * N
    bN = b * N

    pltpu.sync_copy(nb_hbm, nb_v)

    for k in range(RPW // 16):
        c, o = k // (GCH // 16), (k % (GCH // 16)) * 16
        iv = nb_v[pl.ds(i0 + k * 16, 16)]
        i2v = plsc.load_gather(nb_v, [iv])
        idxf_v[c, pl.ds(o, 16)] = iv + bN
        idx2f_v[c, pl.ds(o, 16)] = i2v + bN

    chunks = ([(idxf_v.at[c], xg_hbm, c) for c in range(NCH)]
              + [(idx2f_v.at[c], xg2_hbm, c) for c in range(NCH)])
    bufs = (buf0, buf1)
    sems = (sems0, sems1)
    stores = [None] * len(chunks)
    for j, (idx_ref, out_hbm, c) in enumerate(chunks):
        bi = j & 1
        if j >= 2:
            stores[j - 2].wait()
        pltpu.async_copy(x2d_hbm.at[idx_ref], bufs[bi], semg).wait()
        stores[j] = pltpu.async_copy(
            bufs[bi], out_hbm.at[pl.ds(base + c * GCH, GCH)], sems[bi])
    stores[-2].wait()
    stores[-1].wait()


def _sc_gather(x2d, nb1d):
    mesh = plsc.VectorSubcoreMesh(core_axis_name="c", subcore_axis_name="s")
    f = pl.kernel(
        _sc_gather_body,
        out_type=[jax.ShapeDtypeStruct((BN, C), F32),
                  jax.ShapeDtypeStruct((BN, C), F32)],
        mesh=mesh,
        scratch_types=[
            pltpu.VMEM((N,), jnp.int32),
            pltpu.VMEM((NCH, GCH), jnp.int32),
            pltpu.VMEM((NCH, GCH), jnp.int32),
            pltpu.VMEM((GCH, C), F32),
            pltpu.VMEM((GCH, C), F32),
            pltpu.SemaphoreType.DMA,
            pltpu.SemaphoreType.DMA,
            pltpu.SemaphoreType.DMA,
        ],
        compiler_params=pltpu.CompilerParams(needs_layout_passes=False),
    )
    return f(x2d, nb1d)


def _main_body(x_ref, xg_ref, xg2_ref, wq_ref, wk_ref, wv_ref, wo_ref,
               bq_ref, bk_ref, bv_ref, bo_ref, rb_ref, out_ref,
               m_s, wvo_s, u_s, w_s, bvo_s, c1_s, cs_s, so_s, e_s):
    p = pl.program_id(0)
    i = pl.program_id(1)
    b = i // NPB

    @pl.when((p == 0) & (i == 0))
    def _weights():
        wq = wq_ref[...]
        wk = wk_ref[...]
        wo = wo_ref[...]
        m_s[...] = lax.dot_general(
            wq, wk, (((0,), (0,)), ((), ())),
            preferred_element_type=F32).astype(BF16)
        wvo_s[...] = lax.dot_general(
            wo, wv_ref[...], (((1,), (0,)), ((), ())),
            preferred_element_type=F32).astype(BF16)
        u_s[...] = lax.dot_general(
            bk_ref[...], wq, (((1,), (0,)), ((), ())),
            preferred_element_type=F32)
        w_s[...] = lax.dot_general(
            bq_ref[...], wk, (((1,), (0,)), ((), ())),
            preferred_element_type=F32)
        bvo_s[...] = lax.dot_general(
            bv_ref[...], wo, (((1,), (1,)), ((), ())),
            preferred_element_type=F32)
        c1_s[...] = (jnp.sum(bq_ref[...] * bk_ref[...], axis=1,
                             keepdims=True) + rb_ref[...])
        cs_s[...] = jnp.zeros_like(cs_s)

    @pl.when(p == 0)
    def _scores():
        xb = x_ref[...]
        xgb = xg_ref[...]
        cs_s[pl.ds(b, 1), :] += jnp.sum(xgb, axis=0, keepdims=True)
        pm = lax.dot_general(
            xb.astype(BF16), m_s[...], (((1,), (0,)), ((), ())),
            preferred_element_type=F32)
        s = (jnp.sum(pm * xgb, axis=1, keepdims=True)
             + jnp.sum(xb * u_s[...], axis=1, keepdims=True)
             + jnp.sum(xgb * w_s[...], axis=1, keepdims=True)
             + c1_s[0, 0])
        e_s[pl.ds(i * TM, TM), :] = jnp.exp(
            jnp.minimum(s * (1.0 / math.sqrt(C)), 80.0))

    @pl.when((p == 1) & (i == 0))
    def _so():
        so_s[...] = lax.dot_general(
            cs_s[...].astype(BF16), wvo_s[...], (((1,), (1,)), ((), ())),
            preferred_element_type=F32) + float(N) * bvo_s[...]

    @pl.when(p == 1)
    def _combine():
        g2o = lax.dot_general(
            xg2_ref[...].astype(BF16), wvo_s[...], (((1,), (1,)), ((), ())),
            preferred_element_type=F32) + bvo_s[...]
        e = e_s[pl.ds(i * TM, TM), :]
        z = e + (N - 1.0)
        sob = so_s[pl.ds(b, 1), :]
        out_ref[...] = (sob + (e - 1.0) * g2o) / z + bo_ref[...]


def _pin2(shape):
    return pl.BlockSpec(shape, lambda p, i: (0, 0))


def kernel(x, neighbors, Wq, bq, Wk, bk, Wv, bv, relative_bias, Wo, bo):
    x2d = x.reshape(BN, C)
    nb1d = neighbors[:, 0]

    xg, xg2 = _sc_gather(x2d, nb1d)

    row_p0 = pl.BlockSpec(
        (TM, C), lambda p, i: (jnp.where(p == 0, i, NBLK - 1), 0))
    row_p1 = pl.BlockSpec(
        (TM, C), lambda p, i: (jnp.where(p == 0, 0, i), 0))
    out2d = pl.pallas_call(
        _main_body,
        grid=(2, NBLK),
        in_specs=[row_p0, row_p0, row_p1,
                  _pin2((C, C)), _pin2((C, C)), _pin2((C, C)), _pin2((C, C)),
                  _pin2((1, C)), _pin2((1, C)), _pin2((1, C)), _pin2((1, C)),
                  _pin2((1, 1))],
        out_specs=row_p1,
        out_shape=jax.ShapeDtypeStruct((BN, C), F32),
        scratch_shapes=[pltpu.VMEM((C, C), BF16), pltpu.VMEM((C, C), BF16),
                        pltpu.VMEM((1, C), F32), pltpu.VMEM((1, C), F32),
                        pltpu.VMEM((1, C), F32), pltpu.VMEM((1, 1), F32),
                        pltpu.VMEM((B, C), F32), pltpu.VMEM((B, C), F32),
                        pltpu.VMEM((BN, 1), F32)],
        compiler_params=pltpu.CompilerParams(
            dimension_semantics=("arbitrary", "arbitrary")),
    )(x2d, xg, xg2, Wq, Wk, Wv, Wo, bq.reshape(1, C), bk.reshape(1, C),
      bv.reshape(1, C), bo.reshape(1, C), relative_bias)

    return out2d.reshape(B, N, C)


# speedup vs baseline: 1.1033x; 1.0409x over previous
import math

import jax
import jax.numpy as jnp
from jax import lax
from jax.experimental import pallas as pl
from jax.experimental.pallas import tpu as pltpu
from jax.experimental.pallas import tpu_sc as plsc

B, N, C = 4, 2048, 768
BN = B * N
CH = C // 2
NW = 32
RPW = BN // NW
GCH = 128
NCH = RPW // GCH
TM = 512
NPB = N // TM
NBLK = BN // TM
F32 = jnp.float32
BF16 = jnp.bfloat16
U16 = jnp.uint16
U32 = jnp.uint32


def _pack_halves(lo_bf, hi_bf):
    lo = lax.bitcast_convert_type(lo_bf, U16).astype(U32)
    hi = lax.bitcast_convert_type(hi_bf, U16).astype(U32)
    return lax.bitcast_convert_type(lo | (hi << 16), jnp.int32)


def _unpack_halves(pk_i32):
    v = lax.bitcast_convert_type(pk_i32, U32)
    lo = lax.bitcast_convert_type((v & 0xFFFF).astype(U16), BF16)
    hi = lax.bitcast_convert_type((v >> 16).astype(U16), BF16)
    return lo, hi


def _pack_body(x_ref, xpk_ref):
    xb = x_ref[...].astype(BF16)
    xpk_ref[...] = _pack_halves(xb[:, :CH], xb[:, CH:])


def _sc_gather_body(xpk_hbm, nb_hbm, xg_hbm, xg2_hbm,
                    nb_v, idxf_v, idx2f_v, buf0, buf1,
                    semg, sems0, sems1):
    nc = plsc.get_sparse_core_info().num_cores
    wid = lax.axis_index("s") * nc + lax.axis_index("c")
    base = wid * RPW
    b = base // N
    i0 = base - b * N
    bN = b * N

    pltpu.sync_copy(nb_hbm, nb_v)

    for k in range(RPW // 16):
        c, o = k // (GCH // 16), (k % (GCH // 16)) * 16
        iv = nb_v[pl.ds(i0 + k * 16, 16)]
        i2v = plsc.load_gather(nb_v, [iv])
        idxf_v[c, pl.ds(o, 16)] = iv + bN
        idx2f_v[c, pl.ds(o, 16)] = i2v + bN

    chunks = ([(idxf_v.at[c], xg_hbm, c) for c in range(NCH)]
              + [(idx2f_v.at[c], xg2_hbm, c) for c in range(NCH)])
    bufs = (buf0, buf1)
    sems = (sems0, sems1)
    stores = [None] * len(chunks)
    for j, (idx_ref, out_hbm, c) in enumerate(chunks):
        bi = j & 1
        if j >= 2:
            stores[j - 2].wait()
        pltpu.async_copy(xpk_hbm.at[idx_ref], bufs[bi], semg).wait()
        stores[j] = pltpu.async_copy(
            bufs[bi], out_hbm.at[pl.ds(base + c * GCH, GCH)], sems[bi])
    stores[-2].wait()
    stores[-1].wait()


def _sc_gather(xpk, nb1d):
    mesh = plsc.VectorSubcoreMesh(core_axis_name="c", subcore_axis_name="s")
    f = pl.kernel(
        _sc_gather_body,
        out_type=[jax.ShapeDtypeStruct((BN, CH), jnp.int32),
                  jax.ShapeDtypeStruct((BN, CH), jnp.int32)],
        mesh=mesh,
        scratch_types=[
            pltpu.VMEM((N,), jnp.int32),
            pltpu.VMEM((NCH, GCH), jnp.int32),
            pltpu.VMEM((NCH, GCH), jnp.int32),
            pltpu.VMEM((GCH, CH), jnp.int32),
            pltpu.VMEM((GCH, CH), jnp.int32),
            pltpu.SemaphoreType.DMA,
            pltpu.SemaphoreType.DMA,
            pltpu.SemaphoreType.DMA,
        ],
        compiler_params=pltpu.CompilerParams(needs_layout_passes=False),
    )
    return f(xpk, nb1d)


def _main_body(xpk_ref, xg_ref, xg2_ref, wq_ref, wk_ref, wv_ref, wo_ref,
               bq_ref, bk_ref, bv_ref, bo_ref, rb_ref, out_ref,
               m_s, wvo_s, u_s, w_s, bvo_s, c1_s, cs_s, so_s, e_s):
    p = pl.program_id(0)
    i = pl.program_id(1)
    b = i // NPB

    @pl.when((p == 0) & (i == 0))
    def _weights():
        wq = wq_ref[...]
        wk = wk_ref[...]
        wo = wo_ref[...]
        m_s[...] = lax.dot_general(
            wq, wk, (((0,), (0,)), ((), ())),
            preferred_element_type=F32).astype(BF16)
        wvo_s[...] = lax.dot_general(
            wo, wv_ref[...], (((1,), (0,)), ((), ())),
            preferred_element_type=F32).astype(BF16)
        u_s[...] = lax.dot_general(
            bk_ref[...], wq, (((1,), (0,)), ((), ())),
            preferred_element_type=F32)
        w_s[...] = lax.dot_general(
            bq_ref[...], wk, (((1,), (0,)), ((), ())),
            preferred_element_type=F32)
        bvo_s[...] = lax.dot_general(
            bv_ref[...], wo, (((1,), (1,)), ((), ())),
            preferred_element_type=F32)
        c1_s[...] = (jnp.sum(bq_ref[...] * bk_ref[...], axis=1,
                             keepdims=True) + rb_ref[...])
        cs_s[...] = jnp.zeros_like(cs_s)

    @pl.when(p == 0)
    def _scores():
        xlo, xhi = _unpack_halves(xpk_ref[...])
        glo, ghi = _unpack_halves(xg_ref[...])
        glo32 = glo.astype(F32)
        ghi32 = ghi.astype(F32)
        cs_s[pl.ds(b, 1), :CH] += jnp.sum(glo32, axis=0, keepdims=True)
        cs_s[pl.ds(b, 1), CH:] += jnp.sum(ghi32, axis=0, keepdims=True)
        pm = (lax.dot_general(xlo, m_s[:CH, :], (((1,), (0,)), ((), ())),
                              preferred_element_type=F32)
              + lax.dot_general(xhi, m_s[CH:, :], (((1,), (0,)), ((), ())),
                                preferred_element_type=F32))
        s = (jnp.sum(pm[:, :CH] * glo32, axis=1, keepdims=True)
             + jnp.sum(pm[:, CH:] * ghi32, axis=1, keepdims=True)
             + jnp.sum(xlo.astype(F32) * u_s[:, :CH], axis=1, keepdims=True)
             + jnp.sum(xhi.astype(F32) * u_s[:, CH:], axis=1, keepdims=True)
             + jnp.sum(glo32 * w_s[:, :CH], axis=1, keepdims=True)
             + jnp.sum(ghi32 * w_s[:, CH:], axis=1, keepdims=True)
             + c1_s[0, 0])
        e_s[pl.ds(i * TM, TM), :] = jnp.exp(
            jnp.minimum(s * (1.0 / math.sqrt(C)), 80.0))

    @pl.when((p == 1) & (i == 0))
    def _so():
        so_s[...] = lax.dot_general(
            cs_s[...].astype(BF16), wvo_s[...], (((1,), (1,)), ((), ())),
            preferred_element_type=F32) + float(N) * bvo_s[...]

    @pl.when(p == 1)
    def _combine():
        g2lo, g2hi = _unpack_halves(xg2_ref[...])
        g2o = (lax.dot_general(g2lo, wvo_s[:, :CH], (((1,), (1,)), ((), ())),
                               preferred_element_type=F32)
               + lax.dot_general(g2hi, wvo_s[:, CH:], (((1,), (1,)), ((), ())),
                                 preferred_element_type=F32)
               + bvo_s[...])
        e = e_s[pl.ds(i * TM, TM), :]
        z = e + (N - 1.0)
        sob = so_s[pl.ds(b, 1), :]
        out_ref[...] = (sob + (e - 1.0) * g2o) / z + bo_ref[...]


def _pin2(shape):
    return pl.BlockSpec(shape, lambda p, i: (0, 0))


def kernel(x, neighbors, Wq, bq, Wk, bk, Wv, bv, relative_bias, Wo, bo):
    x2d = x.reshape(BN, C)
    nb1d = neighbors[:, 0]

    xpk = pl.pallas_call(
        _pack_body,
        grid=(NBLK,),
        in_specs=[pl.BlockSpec((TM, C), lambda i: (i, 0))],
        out_specs=pl.BlockSpec((TM, CH), lambda i: (i, 0)),
        out_shape=jax.ShapeDtypeStruct((BN, CH), jnp.int32),
        compiler_params=pltpu.CompilerParams(
            dimension_semantics=("arbitrary",)),
    )(x2d)

    xg, xg2 = _sc_gather(xpk, nb1d)

    row_p0 = pl.BlockSpec(
        (TM, CH), lambda p, i: (jnp.where(p == 0, i, NBLK - 1), 0))
    row_p1pk = pl.BlockSpec(
        (TM, CH), lambda p, i: (jnp.where(p == 0, 0, i), 0))
    out_p1 = pl.BlockSpec(
        (TM, C), lambda p, i: (jnp.where(p == 0, 0, i), 0))
    out2d = pl.pallas_call(
        _main_body,
        grid=(2, NBLK),
        in_specs=[row_p0, row_p0, row_p1pk,
                  _pin2((C, C)), _pin2((C, C)), _pin2((C, C)), _pin2((C, C)),
                  _pin2((1, C)), _pin2((1, C)), _pin2((1, C)), _pin2((1, C)),
                  _pin2((1, 1))],
        out_specs=out_p1,
        out_shape=jax.ShapeDtypeStruct((BN, C), F32),
        scratch_shapes=[pltpu.VMEM((C, C), BF16), pltpu.VMEM((C, C), BF16),
                        pltpu.VMEM((1, C), F32), pltpu.VMEM((1, C), F32),
                        pltpu.VMEM((1, C), F32), pltpu.VMEM((1, 1), F32),
                        pltpu.VMEM((B, C), F32), pltpu.VMEM((B, C), F32),
                        pltpu.VMEM((BN, 1), F32)],
        compiler_params=pltpu.CompilerParams(
            dimension_semantics=("arbitrary", "arbitrary")),
    )(xpk, xg, xg2, Wq, Wk, Wv, Wo, bq.reshape(1, C), bk.reshape(1, C),
      bv.reshape(1, C), bo.reshape(1, C), relative_bias)

    return out2d.reshape(B, N, C)


# speedup vs baseline: 1.1650x; 1.0559x over previous
import math

import jax
import jax.numpy as jnp
from jax import lax
from jax.experimental import pallas as pl
from jax.experimental.pallas import tpu as pltpu
from jax.experimental.pallas import tpu_sc as plsc

B, N, C = 4, 2048, 768
BN = B * N
CH = C // 2
NW = 32
RPW = BN // NW
GCH = 128
NCH = RPW // GCH
TM = 1024
NPB = N // TM
NBLK = BN // TM
F32 = jnp.float32
BF16 = jnp.bfloat16
U16 = jnp.uint16
U32 = jnp.uint32


def _pack_halves(lo_bf, hi_bf):
    lo = lax.bitcast_convert_type(lo_bf, U16).astype(U32)
    hi = lax.bitcast_convert_type(hi_bf, U16).astype(U32)
    return lax.bitcast_convert_type(lo | (hi << 16), jnp.int32)


def _unpack_halves(pk_i32):
    v = lax.bitcast_convert_type(pk_i32, U32)
    lo = lax.bitcast_convert_type((v & 0xFFFF).astype(U16), BF16)
    hi = lax.bitcast_convert_type((v >> 16).astype(U16), BF16)
    return lo, hi


def _pack_body(x_ref, xpk_ref):
    xb = x_ref[...].astype(BF16)
    xpk_ref[...] = _pack_halves(xb[:, :CH], xb[:, CH:])


def _sc_gather_body(xpk_hbm, nb_hbm, xg_hbm, xg2_hbm,
                    nb_v, idxf_v, idx2f_v, buf0, buf1,
                    semg, sems0, sems1):
    nc = plsc.get_sparse_core_info().num_cores
    wid = lax.axis_index("s") * nc + lax.axis_index("c")
    base = wid * RPW
    b = base // N
    i0 = base - b * N
    bN = b * N

    pltpu.sync_copy(nb_hbm, nb_v)

    for k in range(RPW // 16):
        c, o = k // (GCH // 16), (k % (GCH // 16)) * 16
        iv = nb_v[pl.ds(i0 + k * 16, 16)]
        i2v = plsc.load_gather(nb_v, [iv])
        idxf_v[c, pl.ds(o, 16)] = iv + bN
        idx2f_v[c, pl.ds(o, 16)] = i2v + bN

    chunks = ([(idxf_v.at[c], xg_hbm, c) for c in range(NCH)]
              + [(idx2f_v.at[c], xg2_hbm, c) for c in range(NCH)])
    bufs = (buf0, buf1)
    sems = (sems0, sems1)
    stores = [None] * len(chunks)
    for j, (idx_ref, out_hbm, c) in enumerate(chunks):
        bi = j & 1
        if j >= 2:
            stores[j - 2].wait()
        pltpu.async_copy(xpk_hbm.at[idx_ref], bufs[bi], semg).wait()
        stores[j] = pltpu.async_copy(
            bufs[bi], out_hbm.at[pl.ds(base + c * GCH, GCH)], sems[bi])
    stores[-2].wait()
    stores[-1].wait()


def _sc_gather(xpk, nb1d):
    mesh = plsc.VectorSubcoreMesh(core_axis_name="c", subcore_axis_name="s")
    f = pl.kernel(
        _sc_gather_body,
        out_type=[jax.ShapeDtypeStruct((BN, CH), jnp.int32),
                  jax.ShapeDtypeStruct((BN, CH), jnp.int32)],
        mesh=mesh,
        scratch_types=[
            pltpu.VMEM((N,), jnp.int32),
            pltpu.VMEM((NCH, GCH), jnp.int32),
            pltpu.VMEM((NCH, GCH), jnp.int32),
            pltpu.VMEM((GCH, CH), jnp.int32),
            pltpu.VMEM((GCH, CH), jnp.int32),
            pltpu.SemaphoreType.DMA,
            pltpu.SemaphoreType.DMA,
            pltpu.SemaphoreType.DMA,
        ],
        compiler_params=pltpu.CompilerParams(needs_layout_passes=False),
    )
    return f(xpk, nb1d)


def _main_body(xpk_ref, xg_ref, xg2_ref, wq_ref, wk_ref, wv_ref, wo_ref,
               bq_ref, bk_ref, bv_ref, bo_ref, rb_ref, out_ref,
               m_s, wvo_s, u_s, w_s, bvo_s, c1_s, cs_s, so_s, e_s):
    p = pl.program_id(0)
    i = pl.program_id(1)
    b = i // NPB

    @pl.when((p == 0) & (i == 0))
    def _weights():
        wq = wq_ref[...].astype(BF16)
        wk = wk_ref[...].astype(BF16)
        wo = wo_ref[...].astype(BF16)
        m_s[...] = lax.dot_general(
            wq, wk, (((0,), (0,)), ((), ())),
            preferred_element_type=F32).astype(BF16)
        wvo_s[...] = lax.dot_general(
            wo, wv_ref[...].astype(BF16), (((1,), (0,)), ((), ())),
            preferred_element_type=F32).astype(BF16)
        u_s[...] = lax.dot_general(
            bk_ref[...].astype(BF16), wq, (((1,), (0,)), ((), ())),
            preferred_element_type=F32).astype(BF16)
        w_s[...] = lax.dot_general(
            bq_ref[...].astype(BF16), wk, (((1,), (0,)), ((), ())),
            preferred_element_type=F32).astype(BF16)
        bvo_s[...] = lax.dot_general(
            bv_ref[...].astype(BF16), wo, (((1,), (1,)), ((), ())),
            preferred_element_type=F32)
        c1_s[...] = (jnp.sum(bq_ref[...] * bk_ref[...], axis=1,
                             keepdims=True) + rb_ref[...])
        cs_s[...] = jnp.zeros_like(cs_s)

    @pl.when(p == 0)
    def _scores():
        xlo, xhi = _unpack_halves(xpk_ref[...])
        glo, ghi = _unpack_halves(xg_ref[...])
        glo32 = glo.astype(F32)
        ghi32 = ghi.astype(F32)
        cs_s[pl.ds(b, 1), :CH] += jnp.sum(glo32, axis=0, keepdims=True)
        cs_s[pl.ds(b, 1), CH:] += jnp.sum(ghi32, axis=0, keepdims=True)
        pm = (lax.dot_general(xlo, m_s[:CH, :], (((1,), (0,)), ((), ())),
                              preferred_element_type=F32)
              + lax.dot_general(xhi, m_s[CH:, :], (((1,), (0,)), ((), ())),
                                preferred_element_type=F32))
        ul = u_s[:, :CH].astype(F32)
        uh = u_s[:, CH:].astype(F32)
        wl = w_s[:, :CH].astype(F32)
        wh = w_s[:, CH:].astype(F32)
        rd = (pm[:, :CH] * glo32 + pm[:, CH:] * ghi32
              + xlo.astype(F32) * ul + xhi.astype(F32) * uh
              + glo32 * wl + ghi32 * wh)
        s = jnp.sum(rd, axis=1, keepdims=True) + c1_s[0, 0]
        e_s[pl.ds(i * TM, TM), :] = jnp.exp(
            jnp.minimum(s * (1.0 / math.sqrt(C)), 80.0))

    @pl.when((p == 1) & (i == 0))
    def _so():
        so_s[...] = lax.dot_general(
            cs_s[...].astype(BF16), wvo_s[...], (((1,), (1,)), ((), ())),
            preferred_element_type=F32) + float(N) * bvo_s[...]

    @pl.when(p == 1)
    def _combine():
        g2lo, g2hi = _unpack_halves(xg2_ref[...])
        g2o = (lax.dot_general(g2lo, wvo_s[:, :CH], (((1,), (1,)), ((), ())),
                               preferred_element_type=F32)
               + lax.dot_general(g2hi, wvo_s[:, CH:], (((1,), (1,)), ((), ())),
                                 preferred_element_type=F32)
               + bvo_s[...])
        e = e_s[pl.ds(i * TM, TM), :]
        z = e + (N - 1.0)
        sob = so_s[pl.ds(b, 1), :]
        out_ref[...] = (sob + (e - 1.0) * g2o) / z + bo_ref[...]


def _pin2(shape):
    return pl.BlockSpec(shape, lambda p, i: (0, 0))


def kernel(x, neighbors, Wq, bq, Wk, bk, Wv, bv, relative_bias, Wo, bo):
    x2d = x.reshape(BN, C)
    nb1d = neighbors[:, 0]

    xpk = pl.pallas_call(
        _pack_body,
        grid=(NBLK,),
        in_specs=[pl.BlockSpec((TM, C), lambda i: (i, 0))],
        out_specs=pl.BlockSpec((TM, CH), lambda i: (i, 0)),
        out_shape=jax.ShapeDtypeStruct((BN, CH), jnp.int32),
        compiler_params=pltpu.CompilerParams(
            dimension_semantics=("arbitrary",)),
    )(x2d)

    xg, xg2 = _sc_gather(xpk, nb1d)

    row_p0 = pl.BlockSpec(
        (TM, CH), lambda p, i: (jnp.where(p == 0, i, NBLK - 1), 0))
    row_p1pk = pl.BlockSpec(
        (TM, CH), lambda p, i: (jnp.where(p == 0, 0, i), 0))
    out_p1 = pl.BlockSpec(
        (TM, C), lambda p, i: (jnp.where(p == 0, 0, i), 0))
    out2d = pl.pallas_call(
        _main_body,
        grid=(2, NBLK),
        in_specs=[row_p0, row_p0, row_p1pk,
                  _pin2((C, C)), _pin2((C, C)), _pin2((C, C)), _pin2((C, C)),
                  _pin2((1, C)), _pin2((1, C)), _pin2((1, C)), _pin2((1, C)),
                  _pin2((1, 1))],
        out_specs=out_p1,
        out_shape=jax.ShapeDtypeStruct((BN, C), F32),
        scratch_shapes=[pltpu.VMEM((C, C), BF16), pltpu.VMEM((C, C), BF16),
                        pltpu.VMEM((1, C), BF16), pltpu.VMEM((1, C), BF16),
                        pltpu.VMEM((1, C), F32), pltpu.VMEM((1, 1), F32),
                        pltpu.VMEM((B, C), F32), pltpu.VMEM((B, C), F32),
                        pltpu.VMEM((BN, 1), F32)],
        compiler_params=pltpu.CompilerParams(
            dimension_semantics=("arbitrary", "arbitrary")),
    )(xpk, xg, xg2, Wq, Wk, Wv, Wo, bq.reshape(1, C), bk.reshape(1, C),
      bv.reshape(1, C), bo.reshape(1, C), relative_bias)

    return out2d.reshape(B, N, C)


# speedup vs baseline: 1.1715x; 1.0056x over previous
import math

import jax
import jax.numpy as jnp
from jax import lax
from jax.experimental import pallas as pl
from jax.experimental.pallas import tpu as pltpu
from jax.experimental.pallas import tpu_sc as plsc

B, N, C = 4, 2048, 768
BN = B * N
CH = C // 2
NW = 32
RPW = BN // NW
GCH = 128
NCH = RPW // GCH
TM = 2048
NPB = N // TM
NBLK = BN // TM
F32 = jnp.float32
BF16 = jnp.bfloat16
U16 = jnp.uint16
U32 = jnp.uint32


def _pack_halves(lo_bf, hi_bf):
    lo = lax.bitcast_convert_type(lo_bf, U16).astype(U32)
    hi = lax.bitcast_convert_type(hi_bf, U16).astype(U32)
    return lax.bitcast_convert_type(lo | (hi << 16), jnp.int32)


def _unpack_halves(pk_i32):
    v = lax.bitcast_convert_type(pk_i32, U32)
    lo = lax.bitcast_convert_type((v & 0xFFFF).astype(U16), BF16)
    hi = lax.bitcast_convert_type((v >> 16).astype(U16), BF16)
    return lo, hi


def _pack_body(x_ref, xpk_ref):
    xb = x_ref[...].astype(BF16)
    xpk_ref[...] = _pack_halves(xb[:, :CH], xb[:, CH:])


def _sc_gather_body(xpk_hbm, nb_hbm, xg_hbm, xg2_hbm,
                    nb_v, idxf_v, idx2f_v, buf0, buf1,
                    semg, sems0, sems1):
    nc = plsc.get_sparse_core_info().num_cores
    wid = lax.axis_index("s") * nc + lax.axis_index("c")
    base = wid * RPW
    b = base // N
    i0 = base - b * N
    bN = b * N

    pltpu.sync_copy(nb_hbm, nb_v)

    for k in range(RPW // 16):
        c, o = k // (GCH // 16), (k % (GCH // 16)) * 16
        iv = nb_v[pl.ds(i0 + k * 16, 16)]
        i2v = plsc.load_gather(nb_v, [iv])
        idxf_v[c, pl.ds(o, 16)] = iv + bN
        idx2f_v[c, pl.ds(o, 16)] = i2v + bN

    chunks = ([(idxf_v.at[c], xg_hbm, c) for c in range(NCH)]
              + [(idx2f_v.at[c], xg2_hbm, c) for c in range(NCH)])
    bufs = (buf0, buf1)
    sems = (sems0, sems1)
    stores = [None] * len(chunks)
    for j, (idx_ref, out_hbm, c) in enumerate(chunks):
        bi = j & 1
        if j >= 2:
            stores[j - 2].wait()
        pltpu.async_copy(xpk_hbm.at[idx_ref], bufs[bi], semg).wait()
        stores[j] = pltpu.async_copy(
            bufs[bi], out_hbm.at[pl.ds(base + c * GCH, GCH)], sems[bi])
    stores[-2].wait()
    stores[-1].wait()


def _sc_gather(xpk, nb1d):
    mesh = plsc.VectorSubcoreMesh(core_axis_name="c", subcore_axis_name="s")
    f = pl.kernel(
        _sc_gather_body,
        out_type=[jax.ShapeDtypeStruct((BN, CH), jnp.int32),
                  jax.ShapeDtypeStruct((BN, CH), jnp.int32)],
        mesh=mesh,
        scratch_types=[
            pltpu.VMEM((N,), jnp.int32),
            pltpu.VMEM((NCH, GCH), jnp.int32),
            pltpu.VMEM((NCH, GCH), jnp.int32),
            pltpu.VMEM((GCH, CH), jnp.int32),
            pltpu.VMEM((GCH, CH), jnp.int32),
            pltpu.SemaphoreType.DMA,
            pltpu.SemaphoreType.DMA,
            pltpu.SemaphoreType.DMA,
        ],
        compiler_params=pltpu.CompilerParams(needs_layout_passes=False),
    )
    return f(xpk, nb1d)


def _main_body(xpk_ref, xg_ref, xg2_ref, wq_ref, wk_ref, wv_ref, wo_ref,
               bq_ref, bk_ref, bv_ref, bo_ref, rb_ref, out_ref,
               m_s, wvo_s, u_s, w_s, bvo_s, c1_s, cs_s, so_s, e_s):
    p = pl.program_id(0)
    i = pl.program_id(1)
    b = i // NPB

    @pl.when((p == 0) & (i == 0))
    def _weights():
        wq = wq_ref[...].astype(BF16)
        wk = wk_ref[...].astype(BF16)
        wo = wo_ref[...].astype(BF16)
        m_s[...] = lax.dot_general(
            wq, wk, (((0,), (0,)), ((), ())),
            preferred_element_type=F32).astype(BF16)
        wvo_s[...] = lax.dot_general(
            wo, wv_ref[...].astype(BF16), (((1,), (0,)), ((), ())),
            preferred_element_type=F32).astype(BF16)
        u_s[...] = lax.dot_general(
            bk_ref[...].astype(BF16), wq, (((1,), (0,)), ((), ())),
            preferred_element_type=F32).astype(BF16)
        w_s[...] = lax.dot_general(
            bq_ref[...].astype(BF16), wk, (((1,), (0,)), ((), ())),
            preferred_element_type=F32).astype(BF16)
        bvo_s[...] = lax.dot_general(
            bv_ref[...].astype(BF16), wo, (((1,), (1,)), ((), ())),
            preferred_element_type=F32)
        c1_s[...] = (jnp.sum(bq_ref[...] * bk_ref[...], axis=1,
                             keepdims=True) + rb_ref[...])
        cs_s[...] = jnp.zeros_like(cs_s)

    @pl.when(p == 0)
    def _scores():
        xlo, xhi = _unpack_halves(xpk_ref[...])
        glo, ghi = _unpack_halves(xg_ref[...])
        glo32 = glo.astype(F32)
        ghi32 = ghi.astype(F32)
        cs_s[pl.ds(b, 1), :CH] += jnp.sum(glo32, axis=0, keepdims=True)
        cs_s[pl.ds(b, 1), CH:] += jnp.sum(ghi32, axis=0, keepdims=True)
        pm = (lax.dot_general(xlo, m_s[:CH, :], (((1,), (0,)), ((), ())),
                              preferred_element_type=F32)
              + lax.dot_general(xhi, m_s[CH:, :], (((1,), (0,)), ((), ())),
                                preferred_element_type=F32))
        ul = u_s[:, :CH].astype(F32)
        uh = u_s[:, CH:].astype(F32)
        wl = w_s[:, :CH].astype(F32)
        wh = w_s[:, CH:].astype(F32)
        rd = (pm[:, :CH] * glo32 + pm[:, CH:] * ghi32
              + xlo.astype(F32) * ul + xhi.astype(F32) * uh
              + glo32 * wl + ghi32 * wh)
        s = jnp.sum(rd, axis=1, keepdims=True) + c1_s[0, 0]
        e_s[pl.ds(i * TM, TM), :] = jnp.exp(
            jnp.minimum(s * (1.0 / math.sqrt(C)), 80.0))

    @pl.when((p == 1) & (i == 0))
    def _so():
        so_s[...] = lax.dot_general(
            cs_s[...].astype(BF16), wvo_s[...], (((1,), (1,)), ((), ())),
            preferred_element_type=F32) + float(N) * bvo_s[...]

    @pl.when(p == 1)
    def _combine():
        g2lo, g2hi = _unpack_halves(xg2_ref[...])
        g2o = (lax.dot_general(g2lo, wvo_s[:, :CH], (((1,), (1,)), ((), ())),
                               preferred_element_type=F32)
               + lax.dot_general(g2hi, wvo_s[:, CH:], (((1,), (1,)), ((), ())),
                                 preferred_element_type=F32)
               + bvo_s[...])
        e = e_s[pl.ds(i * TM, TM), :]
        z = e + (N - 1.0)
        sob = so_s[pl.ds(b, 1), :]
        out_ref[...] = (sob + (e - 1.0) * g2o) / z + bo_ref[...]


def _pin2(shape):
    return pl.BlockSpec(shape, lambda p, i: (0, 0))


def kernel(x, neighbors, Wq, bq, Wk, bk, Wv, bv, relative_bias, Wo, bo):
    x2d = x.reshape(BN, C)
    nb1d = neighbors[:, 0]

    xpk = pl.pallas_call(
        _pack_body,
        grid=(NBLK,),
        in_specs=[pl.BlockSpec((TM, C), lambda i: (i, 0))],
        out_specs=pl.BlockSpec((TM, CH), lambda i: (i, 0)),
        out_shape=jax.ShapeDtypeStruct((BN, CH), jnp.int32),
        compiler_params=pltpu.CompilerParams(
            dimension_semantics=("arbitrary",)),
    )(x2d)

    xg, xg2 = _sc_gather(xpk, nb1d)

    row_p0 = pl.BlockSpec(
        (TM, CH), lambda p, i: (jnp.where(p == 0, i, NBLK - 1), 0))
    row_p1pk = pl.BlockSpec(
        (TM, CH), lambda p, i: (jnp.where(p == 0, 0, i), 0))
    out_p1 = pl.BlockSpec(
        (TM, C), lambda p, i: (jnp.where(p == 0, 0, i), 0))
    out2d = pl.pallas_call(
        _main_body,
        grid=(2, NBLK),
        in_specs=[row_p0, row_p0, row_p1pk,
                  _pin2((C, C)), _pin2((C, C)), _pin2((C, C)), _pin2((C, C)),
                  _pin2((1, C)), _pin2((1, C)), _pin2((1, C)), _pin2((1, C)),
                  _pin2((1, 1))],
        out_specs=out_p1,
        out_shape=jax.ShapeDtypeStruct((BN, C), F32),
        scratch_shapes=[pltpu.VMEM((C, C), BF16), pltpu.VMEM((C, C), BF16),
                        pltpu.VMEM((1, C), BF16), pltpu.VMEM((1, C), BF16),
                        pltpu.VMEM((1, C), F32), pltpu.VMEM((1, 1), F32),
                        pltpu.VMEM((B, C), F32), pltpu.VMEM((B, C), F32),
                        pltpu.VMEM((BN, 1), F32)],
        compiler_params=pltpu.CompilerParams(
            dimension_semantics=("arbitrary", "arbitrary")),
    )(xpk, xg, xg2, Wq, Wk, Wv, Wo, bq.reshape(1, C), bk.reshape(1, C),
      bv.reshape(1, C), bo.reshape(1, C), relative_bias)

    return out2d.reshape(B, N, C)


# speedup vs baseline: 1.1734x; 1.0016x over previous
import math

import jax
import jax.numpy as jnp
from jax import lax
from jax.experimental import pallas as pl
from jax.experimental.pallas import tpu as pltpu
from jax.experimental.pallas import tpu_sc as plsc

B, N, C = 4, 2048, 768
BN = B * N
CH = C // 2
NW = 32
RPW = BN // NW
GCH = 128
NCH = RPW // GCH
TM = 1024
NPB = N // TM
NBLK = BN // TM
F32 = jnp.float32
BF16 = jnp.bfloat16
U16 = jnp.uint16
U32 = jnp.uint32


def _pack_halves(lo_bf, hi_bf):
    lo = lax.bitcast_convert_type(lo_bf, U16).astype(U32)
    hi = lax.bitcast_convert_type(hi_bf, U16).astype(U32)
    return lax.bitcast_convert_type(lo | (hi << 16), jnp.int32)


def _unpack_f32(pk_i32):
    v = lax.bitcast_convert_type(pk_i32, U32)
    lo = lax.bitcast_convert_type(v << 16, F32)
    hi = lax.bitcast_convert_type(v & U32(0xFFFF0000), F32)
    return lo, hi


def _pack_body(x_ref, xpk_ref):
    xb = x_ref[...].astype(BF16)
    xpk_ref[...] = _pack_halves(xb[:, :CH], xb[:, CH:])


def _sc_gather_body(xpk_hbm, nb_hbm, xg_hbm, xg2_hbm,
                    nb_v, idxf_v, idx2f_v, buf0, buf1,
                    semg, sems0, sems1):
    nc = plsc.get_sparse_core_info().num_cores
    wid = lax.axis_index("s") * nc + lax.axis_index("c")
    base = wid * RPW
    b = base // N
    i0 = base - b * N
    bN = b * N

    pltpu.sync_copy(nb_hbm, nb_v)

    for k in range(RPW // 16):
        c, o = k // (GCH // 16), (k % (GCH // 16)) * 16
        iv = nb_v[pl.ds(i0 + k * 16, 16)]
        i2v = plsc.load_gather(nb_v, [iv])
        idxf_v[c, pl.ds(o, 16)] = iv + bN
        idx2f_v[c, pl.ds(o, 16)] = i2v + bN

    chunks = ([(idxf_v.at[c], xg_hbm, c) for c in range(NCH)]
              + [(idx2f_v.at[c], xg2_hbm, c) for c in range(NCH)])
    bufs = (buf0, buf1)
    sems = (sems0, sems1)
    stores = [None] * len(chunks)
    for j, (idx_ref, out_hbm, c) in enumerate(chunks):
        bi = j & 1
        if j >= 2:
            stores[j - 2].wait()
        pltpu.async_copy(xpk_hbm.at[idx_ref], bufs[bi], semg).wait()
        stores[j] = pltpu.async_copy(
            bufs[bi], out_hbm.at[pl.ds(base + c * GCH, GCH)], sems[bi])
    stores[-2].wait()
    stores[-1].wait()


def _sc_gather(xpk, nb1d):
    mesh = plsc.VectorSubcoreMesh(core_axis_name="c", subcore_axis_name="s")
    f = pl.kernel(
        _sc_gather_body,
        out_type=[jax.ShapeDtypeStruct((BN, CH), jnp.int32),
                  jax.ShapeDtypeStruct((BN, CH), jnp.int32)],
        mesh=mesh,
        scratch_types=[
            pltpu.VMEM((N,), jnp.int32),
            pltpu.VMEM((NCH, GCH), jnp.int32),
            pltpu.VMEM((NCH, GCH), jnp.int32),
            pltpu.VMEM((GCH, CH), jnp.int32),
            pltpu.VMEM((GCH, CH), jnp.int32),
            pltpu.SemaphoreType.DMA,
            pltpu.SemaphoreType.DMA,
            pltpu.SemaphoreType.DMA,
        ],
        compiler_params=pltpu.CompilerParams(needs_layout_passes=False),
    )
    return f(xpk, nb1d)


def _main_body(xpk_ref, xg_ref, xg2_ref, wq_ref, wk_ref, wv_ref, wo_ref,
               bq_ref, bk_ref, bv_ref, bo_ref, rb_ref, out_ref,
               m_s, wvo_s, u_s, w_s, bvo_s, c1_s, cs_s, so_s, e_s):
    p = pl.program_id(0)
    i = pl.program_id(1)
    b = i // NPB

    @pl.when((p == 0) & (i == 0))
    def _weights():
        wq = wq_ref[...].astype(BF16)
        wk = wk_ref[...].astype(BF16)
        wo = wo_ref[...].astype(BF16)
        m_s[...] = lax.dot_general(
            wq, wk, (((0,), (0,)), ((), ())),
            preferred_element_type=F32).astype(BF16)
        wvo_s[...] = lax.dot_general(
            wo, wv_ref[...].astype(BF16), (((1,), (0,)), ((), ())),
            preferred_element_type=F32).astype(BF16)
        u_s[...] = lax.dot_general(
            bk_ref[...].astype(BF16), wq, (((1,), (0,)), ((), ())),
            preferred_element_type=F32)
        w_s[...] = lax.dot_general(
            bq_ref[...].astype(BF16), wk, (((1,), (0,)), ((), ())),
            preferred_element_type=F32)
        bvo_s[...] = lax.dot_general(
            bv_ref[...].astype(BF16), wo, (((1,), (1,)), ((), ())),
            preferred_element_type=F32)
        c1_s[...] = (jnp.sum(bq_ref[...] * bk_ref[...], axis=1,
                             keepdims=True) + rb_ref[...])
        cs_s[...] = jnp.zeros_like(cs_s)

    @pl.when(p == 0)
    def _scores():
        xlo32, xhi32 = _unpack_f32(xpk_ref[...])
        glo32, ghi32 = _unpack_f32(xg_ref[...])
        cs_s[pl.ds(b, 1), :CH] += jnp.sum(glo32, axis=0, keepdims=True)
        cs_s[pl.ds(b, 1), CH:] += jnp.sum(ghi32, axis=0, keepdims=True)
        pm = (lax.dot_general(xlo32.astype(BF16), m_s[:CH, :],
                              (((1,), (0,)), ((), ())),
                              preferred_element_type=F32)
              + lax.dot_general(xhi32.astype(BF16), m_s[CH:, :],
                                (((1,), (0,)), ((), ())),
                                preferred_element_type=F32)
              + w_s[...])
        rd = (pm[:, :CH] * glo32 + pm[:, CH:] * ghi32
              + xlo32 * u_s[:, :CH] + xhi32 * u_s[:, CH:])
        s = jnp.sum(rd, axis=1, keepdims=True) + c1_s[0, 0]
        e_s[pl.ds(i * TM, TM), :] = jnp.exp(
            jnp.minimum(s * (1.0 / math.sqrt(C)), 80.0))

    @pl.when((p == 1) & (i == 0))
    def _so():
        so_s[...] = lax.dot_general(
            cs_s[...].astype(BF16), wvo_s[...], (((1,), (1,)), ((), ())),
            preferred_element_type=F32) + float(N) * bvo_s[...]

    @pl.when(p == 1)
    def _combine():
        g2lo32, g2hi32 = _unpack_f32(xg2_ref[...])
        g2o = (lax.dot_general(g2lo32.astype(BF16), wvo_s[:, :CH],
                               (((1,), (1,)), ((), ())),
                               preferred_element_type=F32)
               + lax.dot_general(g2hi32.astype(BF16), wvo_s[:, CH:],
                                 (((1,), (1,)), ((), ())),
                                 preferred_element_type=F32)
               + bvo_s[...])
        e = e_s[pl.ds(i * TM, TM), :]
        z = e + (N - 1.0)
        sob = so_s[pl.ds(b, 1), :]
        out_ref[...] = (sob + (e - 1.0) * g2o) / z + bo_ref[...]


def _pin2(shape):
    return pl.BlockSpec(shape, lambda p, i: (0, 0))


def kernel(x, neighbors, Wq, bq, Wk, bk, Wv, bv, relative_bias, Wo, bo):
    x2d = x.reshape(BN, C)
    nb1d = neighbors[:, 0]

    xpk = pl.pallas_call(
        _pack_body,
        grid=(NBLK,),
        in_specs=[pl.BlockSpec((TM, C), lambda i: (i, 0))],
        out_specs=pl.BlockSpec((TM, CH), lambda i: (i, 0)),
        out_shape=jax.ShapeDtypeStruct((BN, CH), jnp.int32),
        compiler_params=pltpu.CompilerParams(
            dimension_semantics=("arbitrary",)),
    )(x2d)

    xg, xg2 = _sc_gather(xpk, nb1d)

    row_p0 = pl.BlockSpec(
        (TM, CH), lambda p, i: (jnp.where(p == 0, i, NBLK - 1), 0))
    row_p1pk = pl.BlockSpec(
        (TM, CH), lambda p, i: (jnp.where(p == 0, 0, i), 0))
    out_p1 = pl.BlockSpec(
        (TM, C), lambda p, i: (jnp.where(p == 0, 0, i), 0))
    out2d = pl.pallas_call(
        _main_body,
        grid=(2, NBLK),
        in_specs=[row_p0, row_p0, row_p1pk,
                  _pin2((C, C)), _pin2((C, C)), _pin2((C, C)), _pin2((C, C)),
                  _pin2((1, C)), _pin2((1, C)), _pin2((1, C)), _pin2((1, C)),
                  _pin2((1, 1))],
        out_specs=out_p1,
        out_shape=jax.ShapeDtypeStruct((BN, C), F32),
        scratch_shapes=[pltpu.VMEM((C, C), BF16), pltpu.VMEM((C, C), BF16),
                        pltpu.VMEM((1, C), F32), pltpu.VMEM((1, C), F32),
                        pltpu.VMEM((1, C), F32), pltpu.VMEM((1, 1), F32),
                        pltpu.VMEM((B, C), F32), pltpu.VMEM((B, C), F32),
                        pltpu.VMEM((BN, 1), F32)],
        compiler_params=pltpu.CompilerParams(
            dimension_semantics=("arbitrary", "arbitrary")),
    )(xpk, xg, xg2, Wq, Wk, Wv, Wo, bq.reshape(1, C), bk.reshape(1, C),
      bv.reshape(1, C), bo.reshape(1, C), relative_bias)

    return out2d.reshape(B, N, C)
